# Initial kernel scaffold; baseline (speedup 1.0000x reference)
import jax
import jax.numpy as jnp
from jax.experimental import pallas as pl


def kernel(x, y_raw, enc, dec, et, dt, codebook):
    raise NotImplementedError("write your pallas kernel here")


# speedup vs baseline: 1.1368x; 1.1368x over previous
import jax
import jax.numpy as jnp
from jax.experimental import pallas as pl

B = 8
N_ROI = 160
T_LEN = 160
PATCH = 16
N_EMBD = 384
N_HEADS = 6
DH = N_EMBD // N_HEADS
MLP = 1536
EMBED_DIM = 128
K_CODE = 8192
D_OUT = 160
TOK = N_ROI * (T_LEN // PATCH)
ROWS = B * TOK
R_TILE = 1280
CHUNK = 2048

_f32 = jnp.float32


def _lnk(h, g, b):
    m = jnp.mean(h, -1, keepdims=True)
    v = jnp.var(h, -1, keepdims=True)
    return (h - m) / jnp.sqrt(v + 1e-6) * g + b


def _enc_pre_k(x_ref, wp_ref, bp_ref, g1_ref, b1_ref, wqkv_ref, h0_ref, qkv_ref):
    h0 = jnp.dot(x_ref[0], wp_ref[...], preferred_element_type=_f32) + bp_ref[...]
    h0_ref[0] = h0
    ln = _lnk(h0, g1_ref[...], b1_ref[...])
    qkv_ref[0] = jnp.dot(ln, wqkv_ref[...], preferred_element_type=_f32)


def _dec_pre_k(zq_ref, wp_ref, bp_ref, g1_ref, b1_ref, wqkv_ref, h0_ref, qkv_ref):
    h0 = jnp.dot(zq_ref[0], wp_ref[...], preferred_element_type=_f32) + bp_ref[...]
    h0_ref[0] = h0
    ln = _lnk(h0, g1_ref[...], b1_ref[...])
    qkv_ref[0] = jnp.dot(ln, wqkv_ref[...], preferred_element_type=_f32)


def _attn_k(qkv_ref, o_ref):
    qkv = qkv_ref[0]
    scale = 1.0 / jnp.sqrt(_f32(DH))
    parts = []
    for h in range(N_HEADS):
        q = qkv[:, h * DH:(h + 1) * DH]
        k = qkv[:, N_EMBD + h * DH:N_EMBD + (h + 1) * DH]
        v = qkv[:, 2 * N_EMBD + h * DH:2 * N_EMBD + (h + 1) * DH]
        s = jnp.dot(q, k.T, preferred_element_type=_f32) * scale
        p = jax.nn.softmax(s, axis=-1)
        parts.append(jnp.dot(p, v, preferred_element_type=_f32))
    o_ref[0] = jnp.concatenate(parts, axis=1)


def _post_k(o_ref, h0_ref, wo_ref, g2_ref, b2_ref, w1_ref, bm1_ref, w2_ref,
            bm2_ref, ef_ref):
    h1 = h0_ref[0] + jnp.dot(o_ref[0], wo_ref[...], preferred_element_type=_f32)
    ln = _lnk(h1, g2_ref[...], b2_ref[...])
    t = jax.nn.gelu(jnp.dot(ln, w1_ref[...], preferred_element_type=_f32)
                    + bm1_ref[...])
    ef_ref[0] = h1 + jnp.dot(t, w2_ref[...], preferred_element_type=_f32) + bm2_ref[...]


def _toz_k(ef_ref, w1_ref, b1_ref, w2_ref, b2_ref, zf_ref):
    t = jnp.tanh(jnp.dot(ef_ref[0], w1_ref[...], preferred_element_type=_f32)
                 + b1_ref[...])
    z = jnp.dot(t, w2_ref[...], preferred_element_type=_f32) + b2_ref[...]
    n = jnp.sqrt(jnp.sum(z * z, -1, keepdims=True))
    zf_ref[0] = z / jnp.maximum(n, 1e-12)


def _ncb_k(cb_ref, en_ref):
    cb = cb_ref[...]
    n = jnp.sqrt(jnp.sum(cb * cb, -1, keepdims=True))
    en_ref[...] = cb / jnp.maximum(n, 1e-12)


def _vq_k(zf_ref, en_ref, zq_ref, emb_ref):
    zf = zf_ref[...]
    z2 = jnp.sum(zf * zf, -1, keepdims=True)
    best_d = jnp.full((R_TILE, 1), jnp.inf, _f32)
    best_i = jnp.zeros((R_TILE, 1), jnp.int32)
    for c in range(K_CODE // CHUNK):
        e = en_ref[pl.ds(c * CHUNK, CHUNK), :]
        e2 = jnp.sum(e * e, -1)[None, :]
        s = jnp.dot(zf, e.T, preferred_element_type=_f32)
        d = z2 + e2 - 2.0 * s
        ld = jnp.min(d, axis=1, keepdims=True)
        li = jnp.argmin(d, axis=1).astype(jnp.int32)[:, None]
        upd = ld < best_d
        best_i = jnp.where(upd, li + c * CHUNK, best_i)
        best_d = jnp.where(upd, ld, best_d)
    zq = jnp.zeros((R_TILE, EMBED_DIM), _f32)
    for c in range(K_CODE // CHUNK):
        e = en_ref[pl.ds(c * CHUNK, CHUNK), :]
        cols = jax.lax.broadcasted_iota(jnp.int32, (R_TILE, CHUNK), 1) + c * CHUNK
        oh = (best_i == cols).astype(_f32)
        zq = zq + jnp.dot(oh, e, preferred_element_type=_f32,
                          precision=jax.lax.Precision.HIGHEST)
    zq_ref[...] = zq
    df = zq - zf
    emb_ref[0] = jnp.sum(df * df, axis=0)[None, :]


def _rec_k(df_ref, y_ref, w1_ref, b1_ref, w2_ref, b2_ref, out_ref):
    t = jnp.tanh(jnp.dot(df_ref[0], w1_ref[...], preferred_element_type=_f32)
                 + b1_ref[...])
    rec = jnp.dot(t, w2_ref[...], preferred_element_type=_f32) + b2_ref[...]
    r = rec - y_ref[0]
    out_ref[0] = jnp.sum(r * r, axis=0)[None, :]


def _full(shape):
    n = len(shape)
    return pl.BlockSpec(shape, lambda *_: (0,) * n)


def _enc_pre(x, wp, bp, g1, b1, wqkv):
    return pl.pallas_call(
        _enc_pre_k,
        grid=(B,),
        in_specs=[
            pl.BlockSpec((1, TOK, PATCH), lambda b: (b, 0, 0)),
            _full((PATCH, N_EMBD)), _full((1, N_EMBD)), _full((1, N_EMBD)),
            _full((1, N_EMBD)), _full((N_EMBD, 3 * N_EMBD)),
        ],
        out_specs=[
            pl.BlockSpec((1, TOK, N_EMBD), lambda b: (b, 0, 0)),
            pl.BlockSpec((1, TOK, 3 * N_EMBD), lambda b: (b, 0, 0)),
        ],
        out_shape=[
            jax.ShapeDtypeStruct((B, TOK, N_EMBD), _f32),
            jax.ShapeDtypeStruct((B, TOK, 3 * N_EMBD), _f32),
        ],
    )(x, wp, bp, g1, b1, wqkv)


def _dec_pre(zq, wp, bp, g1, b1, wqkv):
    return pl.pallas_call(
        _dec_pre_k,
        grid=(B,),
        in_specs=[
            pl.BlockSpec((1, TOK, EMBED_DIM), lambda b: (b, 0, 0)),
            _full((EMBED_DIM, N_EMBD)), _full((1, N_EMBD)), _full((1, N_EMBD)),
            _full((1, N_EMBD)), _full((N_EMBD, 3 * N_EMBD)),
        ],
        out_specs=[
            pl.BlockSpec((1, TOK, N_EMBD), lambda b: (b, 0, 0)),
            pl.BlockSpec((1, TOK, 3 * N_EMBD), lambda b: (b, 0, 0)),
        ],
        out_shape=[
            jax.ShapeDtypeStruct((B, TOK, N_EMBD), _f32),
            jax.ShapeDtypeStruct((B, TOK, 3 * N_EMBD), _f32),
        ],
    )(zq, wp, bp, g1, b1, wqkv)


def _attn(qkv):
    return pl.pallas_call(
        _attn_k,
        grid=(B,),
        in_specs=[
            pl.BlockSpec((1, TOK, 3 * N_EMBD), lambda b: (b, 0, 0)),
        ],
        out_specs=pl.BlockSpec((1, TOK, N_EMBD), lambda b: (b, 0, 0)),
        out_shape=jax.ShapeDtypeStruct((B, TOK, N_EMBD), _f32),
    )(qkv)


def _post(o, h0, wo, g2, b2, w1, bm1, w2, bm2):
    return pl.pallas_call(
        _post_k,
        grid=(B,),
        in_specs=[
            pl.BlockSpec((1, TOK, N_EMBD), lambda b: (b, 0, 0)),
            pl.BlockSpec((1, TOK, N_EMBD), lambda b: (b, 0, 0)),
            _full((N_EMBD, N_EMBD)), _full((1, N_EMBD)), _full((1, N_EMBD)),
            _full((N_EMBD, MLP)), _full((1, MLP)),
            _full((MLP, N_EMBD)), _full((1, N_EMBD)),
        ],
        out_specs=pl.BlockSpec((1, TOK, N_EMBD), lambda b: (b, 0, 0)),
        out_shape=jax.ShapeDtypeStruct((B, TOK, N_EMBD), _f32),
    )(o, h0, wo, g2, b2, w1, bm1, w2, bm2)


def _toz(ef, w1, b1, w2, b2):
    return pl.pallas_call(
        _toz_k,
        grid=(B,),
        in_specs=[
            pl.BlockSpec((1, TOK, N_EMBD), lambda b: (b, 0, 0)),
            _full((N_EMBD, N_EMBD)), _full((1, N_EMBD)),
            _full((N_EMBD, EMBED_DIM)), _full((1, EMBED_DIM)),
        ],
        out_specs=pl.BlockSpec((1, TOK, EMBED_DIM), lambda b: (b, 0, 0)),
        out_shape=jax.ShapeDtypeStruct((B, TOK, EMBED_DIM), _f32),
    )(ef, w1, b1, w2, b2)


def _ncb(cb):
    return pl.pallas_call(
        _ncb_k,
        in_specs=[_full((K_CODE, EMBED_DIM))],
        out_specs=_full((K_CODE, EMBED_DIM)),
        out_shape=jax.ShapeDtypeStruct((K_CODE, EMBED_DIM), _f32),
    )(cb)


def _vq(zf_flat, en):
    n_tiles = ROWS // R_TILE
    return pl.pallas_call(
        _vq_k,
        grid=(n_tiles,),
        in_specs=[
            pl.BlockSpec((R_TILE, EMBED_DIM), lambda i: (i, 0)),
            _full((K_CODE, EMBED_DIM)),
        ],
        out_specs=[
            pl.BlockSpec((R_TILE, EMBED_DIM), lambda i: (i, 0)),
            pl.BlockSpec((1, 1, EMBED_DIM), lambda i: (i, 0, 0)),
        ],
        out_shape=[
            jax.ShapeDtypeStruct((ROWS, EMBED_DIM), _f32),
            jax.ShapeDtypeStruct((n_tiles, 1, EMBED_DIM), _f32),
        ],
    )(zf_flat, en)


def _rec(df, y_raw, w1, b1, w2, b2):
    return pl.pallas_call(
        _rec_k,
        grid=(B,),
        in_specs=[
            pl.BlockSpec((1, N_ROI, (T_LEN // PATCH) * N_EMBD), lambda b: (b, 0, 0)),
            pl.BlockSpec((1, N_ROI, D_OUT), lambda b: (b, 0, 0)),
            _full(((T_LEN // PATCH) * N_EMBD, N_EMBD)), _full((1, N_EMBD)),
            _full((N_EMBD, D_OUT)), _full((1, D_OUT)),
        ],
        out_specs=pl.BlockSpec((1, 1, D_OUT), lambda b: (b, 0, 0)),
        out_shape=jax.ShapeDtypeStruct((B, 1, D_OUT), _f32),
    )(df, y_raw, w1, b1, w2, b2)


def kernel(x, y_raw, enc, dec, et, dt, codebook):
    r2 = lambda a: a.reshape(1, -1)

    h0, qkv = _enc_pre(x.reshape(B, TOK, PATCH), enc['Wp'], r2(enc['bp']),
                       r2(enc['g1']), r2(enc['b1']), enc['Wqkv'])
    o = _attn(qkv)
    ef = _post(o, h0, enc['Wo'], r2(enc['g2']), r2(enc['b2']),
               enc['W1'], r2(enc['bm1']), enc['W2'], r2(enc['bm2']))

    zf = _toz(ef, et['W1'], r2(et['b1']), et['W2'], r2(et['b2']))
    en = _ncb(codebook)
    zq_flat, emb_part = _vq(zf.reshape(ROWS, EMBED_DIM), en)

    hd0, qkvd = _dec_pre(zq_flat.reshape(B, TOK, EMBED_DIM), dec['Wp'],
                         r2(dec['bp']), r2(dec['g1']), r2(dec['b1']), dec['Wqkv'])
    od = _attn(qkvd)
    df = _post(od, hd0, dec['Wo'], r2(dec['g2']), r2(dec['b2']),
               dec['W1'], r2(dec['bm1']), dec['W2'], r2(dec['bm2']))

    rec_part = _rec(df.reshape(B, N_ROI, (T_LEN // PATCH) * N_EMBD), y_raw,
                    dt['W1'], r2(dt['b1']), dt['W2'], r2(dt['b2']))

    emb_loss = jnp.sum(emb_part) / _f32(ROWS * EMBED_DIM)
    rec_loss = jnp.sum(rec_part) / _f32(B * N_ROI * D_OUT)
    return (emb_loss + rec_loss, ef)


# speedup vs baseline: 1.6782x; 1.4762x over previous
import functools

import jax
import jax.numpy as jnp
from jax.experimental import pallas as pl
from jax.experimental.pallas import tpu as pltpu
from jax.experimental.pallas import tpu_sc as plsc

B = 8
N_ROI = 160
T_LEN = 160
PATCH = 16
N_EMBD = 384
N_HEADS = 6
DH = N_EMBD // N_HEADS
MLP = 1536
EMBED_DIM = 128
K_CODE = 8192
D_OUT = 160
TOK = N_ROI * (T_LEN // PATCH)
ROWS = B * TOK
R_TILE = 1280
CHUNK = 2048

_f32 = jnp.float32


def _lnk(h, g, b):
    m = jnp.mean(h, -1, keepdims=True)
    v = jnp.var(h, -1, keepdims=True)
    return (h - m) / jnp.sqrt(v + 1e-6) * g + b


def _enc_pre_k(x_ref, wp_ref, bp_ref, g1_ref, b1_ref, wqkv_ref, h0_ref, qkv_ref):
    h0 = jnp.dot(x_ref[0], wp_ref[...], preferred_element_type=_f32) + bp_ref[...]
    h0_ref[0] = h0
    ln = _lnk(h0, g1_ref[...], b1_ref[...])
    qkv_ref[0] = jnp.dot(ln, wqkv_ref[...], preferred_element_type=_f32)


def _dec_pre_k(zq_ref, wp_ref, bp_ref, g1_ref, b1_ref, wqkv_ref, h0_ref, qkv_ref):
    h0 = jnp.dot(zq_ref[0], wp_ref[...], preferred_element_type=_f32) + bp_ref[...]
    h0_ref[0] = h0
    ln = _lnk(h0, g1_ref[...], b1_ref[...])
    qkv_ref[0] = jnp.dot(ln, wqkv_ref[...], preferred_element_type=_f32)


def _attn_k(qkv_ref, o_ref):
    qkv = qkv_ref[0]
    scale = 1.0 / jnp.sqrt(_f32(DH))
    parts = []
    for h in range(N_HEADS):
        q = qkv[:, h * DH:(h + 1) * DH]
        k = qkv[:, N_EMBD + h * DH:N_EMBD + (h + 1) * DH]
        v = qkv[:, 2 * N_EMBD + h * DH:2 * N_EMBD + (h + 1) * DH]
        s = jnp.dot(q, k.T, preferred_element_type=_f32) * scale
        p = jax.nn.softmax(s, axis=-1)
        parts.append(jnp.dot(p, v, preferred_element_type=_f32))
    o_ref[0] = jnp.concatenate(parts, axis=1)


def _post_k(o_ref, h0_ref, wo_ref, g2_ref, b2_ref, w1_ref, bm1_ref, w2_ref,
            bm2_ref, ef_ref):
    h1 = h0_ref[0] + jnp.dot(o_ref[0], wo_ref[...], preferred_element_type=_f32)
    ln = _lnk(h1, g2_ref[...], b2_ref[...])
    t = jax.nn.gelu(jnp.dot(ln, w1_ref[...], preferred_element_type=_f32)
                    + bm1_ref[...])
    ef_ref[0] = h1 + jnp.dot(t, w2_ref[...], preferred_element_type=_f32) + bm2_ref[...]


def _toz_k(ef_ref, w1_ref, b1_ref, w2_ref, b2_ref, zf_ref):
    t = jnp.tanh(jnp.dot(ef_ref[0], w1_ref[...], preferred_element_type=_f32)
                 + b1_ref[...])
    z = jnp.dot(t, w2_ref[...], preferred_element_type=_f32) + b2_ref[...]
    n = jnp.sqrt(jnp.sum(z * z, -1, keepdims=True))
    zf_ref[0] = z / jnp.maximum(n, 1e-12)


def _ncb_k(cb_ref, en_ref):
    cb = cb_ref[...]
    n = jnp.sqrt(jnp.sum(cb * cb, -1, keepdims=True))
    en_ref[...] = cb / jnp.maximum(n, 1e-12)


def _vq_k(zf_ref, en_ref, idx_ref, emb_ref):
    zf = zf_ref[...]
    z2 = jnp.sum(zf * zf, -1, keepdims=True)
    best_d = jnp.full((R_TILE, 1), jnp.inf, _f32)
    best_i = jnp.zeros((R_TILE, 1), jnp.int32)
    for c in range(K_CODE // CHUNK):
        e = en_ref[pl.ds(c * CHUNK, CHUNK), :]
        e2 = jnp.sum(e * e, -1)[None, :]
        s = jnp.dot(zf, e.T, preferred_element_type=_f32)
        d = z2 + e2 - 2.0 * s
        ld = jnp.min(d, axis=1, keepdims=True)
        li = jnp.argmin(d, axis=1).astype(jnp.int32)[:, None]
        upd = ld < best_d
        best_i = jnp.where(upd, li + c * CHUNK, best_i)
        best_d = jnp.where(upd, ld, best_d)
    idx_ref[...] = best_i
    emb_ref[0] = jnp.full((1, EMBED_DIM), jnp.sum(best_d) / _f32(EMBED_DIM))


def _rec_k(df_ref, y_ref, w1_ref, b1_ref, w2_ref, b2_ref, out_ref):
    t = jnp.tanh(jnp.dot(df_ref[0], w1_ref[...], preferred_element_type=_f32)
                 + b1_ref[...])
    rec = jnp.dot(t, w2_ref[...], preferred_element_type=_f32) + b2_ref[...]
    r = rec - y_ref[0]
    out_ref[0] = jnp.sum(r * r, axis=0)[None, :]


def _full(shape):
    n = len(shape)
    return pl.BlockSpec(shape, lambda *_: (0,) * n)


def _enc_pre(x, wp, bp, g1, b1, wqkv):
    return pl.pallas_call(
        _enc_pre_k,
        grid=(B,),
        in_specs=[
            pl.BlockSpec((1, TOK, PATCH), lambda b: (b, 0, 0)),
            _full((PATCH, N_EMBD)), _full((1, N_EMBD)), _full((1, N_EMBD)),
            _full((1, N_EMBD)), _full((N_EMBD, 3 * N_EMBD)),
        ],
        out_specs=[
            pl.BlockSpec((1, TOK, N_EMBD), lambda b: (b, 0, 0)),
            pl.BlockSpec((1, TOK, 3 * N_EMBD), lambda b: (b, 0, 0)),
        ],
        out_shape=[
            jax.ShapeDtypeStruct((B, TOK, N_EMBD), _f32),
            jax.ShapeDtypeStruct((B, TOK, 3 * N_EMBD), _f32),
        ],
    )(x, wp, bp, g1, b1, wqkv)


def _dec_pre(zq, wp, bp, g1, b1, wqkv):
    return pl.pallas_call(
        _dec_pre_k,
        grid=(B,),
        in_specs=[
            pl.BlockSpec((1, TOK, EMBED_DIM), lambda b: (b, 0, 0)),
            _full((EMBED_DIM, N_EMBD)), _full((1, N_EMBD)), _full((1, N_EMBD)),
            _full((1, N_EMBD)), _full((N_EMBD, 3 * N_EMBD)),
        ],
        out_specs=[
            pl.BlockSpec((1, TOK, N_EMBD), lambda b: (b, 0, 0)),
            pl.BlockSpec((1, TOK, 3 * N_EMBD), lambda b: (b, 0, 0)),
        ],
        out_shape=[
            jax.ShapeDtypeStruct((B, TOK, N_EMBD), _f32),
            jax.ShapeDtypeStruct((B, TOK, 3 * N_EMBD), _f32),
        ],
    )(zq, wp, bp, g1, b1, wqkv)


def _attn(qkv):
    return pl.pallas_call(
        _attn_k,
        grid=(B,),
        in_specs=[
            pl.BlockSpec((1, TOK, 3 * N_EMBD), lambda b: (b, 0, 0)),
        ],
        out_specs=pl.BlockSpec((1, TOK, N_EMBD), lambda b: (b, 0, 0)),
        out_shape=jax.ShapeDtypeStruct((B, TOK, N_EMBD), _f32),
    )(qkv)


def _post(o, h0, wo, g2, b2, w1, bm1, w2, bm2):
    return pl.pallas_call(
        _post_k,
        grid=(B,),
        in_specs=[
            pl.BlockSpec((1, TOK, N_EMBD), lambda b: (b, 0, 0)),
            pl.BlockSpec((1, TOK, N_EMBD), lambda b: (b, 0, 0)),
            _full((N_EMBD, N_EMBD)), _full((1, N_EMBD)), _full((1, N_EMBD)),
            _full((N_EMBD, MLP)), _full((1, MLP)),
            _full((MLP, N_EMBD)), _full((1, N_EMBD)),
        ],
        out_specs=pl.BlockSpec((1, TOK, N_EMBD), lambda b: (b, 0, 0)),
        out_shape=jax.ShapeDtypeStruct((B, TOK, N_EMBD), _f32),
    )(o, h0, wo, g2, b2, w1, bm1, w2, bm2)


def _toz(ef, w1, b1, w2, b2):
    return pl.pallas_call(
        _toz_k,
        grid=(B,),
        in_specs=[
            pl.BlockSpec((1, TOK, N_EMBD), lambda b: (b, 0, 0)),
            _full((N_EMBD, N_EMBD)), _full((1, N_EMBD)),
            _full((N_EMBD, EMBED_DIM)), _full((1, EMBED_DIM)),
        ],
        out_specs=pl.BlockSpec((1, TOK, EMBED_DIM), lambda b: (b, 0, 0)),
        out_shape=jax.ShapeDtypeStruct((B, TOK, EMBED_DIM), _f32),
    )(ef, w1, b1, w2, b2)


def _ncb(cb):
    return pl.pallas_call(
        _ncb_k,
        in_specs=[_full((K_CODE, EMBED_DIM))],
        out_specs=_full((K_CODE, EMBED_DIM)),
        out_shape=jax.ShapeDtypeStruct((K_CODE, EMBED_DIM), _f32),
    )(cb)


def _vq(zf_flat, en):
    n_tiles = ROWS // R_TILE
    return pl.pallas_call(
        _vq_k,
        grid=(n_tiles,),
        in_specs=[
            pl.BlockSpec((R_TILE, EMBED_DIM), lambda i: (i, 0)),
            _full((K_CODE, EMBED_DIM)),
        ],
        out_specs=[
            pl.BlockSpec((R_TILE, 1), lambda i: (i, 0)),
            pl.BlockSpec((1, 1, EMBED_DIM), lambda i: (i, 0, 0)),
        ],
        out_shape=[
            jax.ShapeDtypeStruct((ROWS, 1), jnp.int32),
            jax.ShapeDtypeStruct((n_tiles, 1, EMBED_DIM), _f32),
        ],
    )(zf_flat, en)


_SC_NW = 32
_SC_BPW = ROWS // _SC_NW
_SC_CH = 80
_SC_NCH = _SC_BPW // _SC_CH


def _sc_gather(table, idx):
    mesh = plsc.VectorSubcoreMesh(core_axis_name="c", subcore_axis_name="s")

    @functools.partial(
        pl.kernel, mesh=mesh,
        out_type=jax.ShapeDtypeStruct((ROWS, EMBED_DIM), _f32),
        scratch_types=[
            pltpu.VMEM((_SC_NCH, _SC_CH), jnp.int32),
            pltpu.VMEM((_SC_NCH, _SC_CH, EMBED_DIM), _f32),
            pltpu.SemaphoreType.DMA,
        ],
    )
    def k(table_hbm, idx_hbm, out_hbm, idx_v, rows_v, sem):
        wid = jax.lax.axis_index("s") * 2 + jax.lax.axis_index("c")
        base = wid * _SC_BPW
        for j in range(_SC_NCH):
            pltpu.sync_copy(idx_hbm.at[pl.ds(base + j * _SC_CH, _SC_CH)],
                            idx_v.at[j])
        for j in range(_SC_NCH):
            pltpu.async_copy(table_hbm.at[idx_v.at[j]], rows_v.at[j], sem).wait()
        for j in range(_SC_NCH):
            pltpu.sync_copy(rows_v.at[j],
                            out_hbm.at[pl.ds(base + j * _SC_CH, _SC_CH)])

    return k(table, idx)


def _rec(df, y_raw, w1, b1, w2, b2):
    return pl.pallas_call(
        _rec_k,
        grid=(B,),
        in_specs=[
            pl.BlockSpec((1, N_ROI, (T_LEN // PATCH) * N_EMBD), lambda b: (b, 0, 0)),
            pl.BlockSpec((1, N_ROI, D_OUT), lambda b: (b, 0, 0)),
            _full(((T_LEN // PATCH) * N_EMBD, N_EMBD)), _full((1, N_EMBD)),
            _full((N_EMBD, D_OUT)), _full((1, D_OUT)),
        ],
        out_specs=pl.BlockSpec((1, 1, D_OUT), lambda b: (b, 0, 0)),
        out_shape=jax.ShapeDtypeStruct((B, 1, D_OUT), _f32),
    )(df, y_raw, w1, b1, w2, b2)


def kernel(x, y_raw, enc, dec, et, dt, codebook):
    r2 = lambda a: a.reshape(1, -1)

    h0, qkv = _enc_pre(x.reshape(B, TOK, PATCH), enc['Wp'], r2(enc['bp']),
                       r2(enc['g1']), r2(enc['b1']), enc['Wqkv'])
    o = _attn(qkv)
    ef = _post(o, h0, enc['Wo'], r2(enc['g2']), r2(enc['b2']),
               enc['W1'], r2(enc['bm1']), enc['W2'], r2(enc['bm2']))

    zf = _toz(ef, et['W1'], r2(et['b1']), et['W2'], r2(et['b2']))
    en = _ncb(codebook)
    idx, emb_part = _vq(zf.reshape(ROWS, EMBED_DIM), en)
    zq_flat = _sc_gather(en, idx.reshape(ROWS))

    hd0, qkvd = _dec_pre(zq_flat.reshape(B, TOK, EMBED_DIM), dec['Wp'],
                         r2(dec['bp']), r2(dec['g1']), r2(dec['b1']), dec['Wqkv'])
    od = _attn(qkvd)
    df = _post(od, hd0, dec['Wo'], r2(dec['g2']), r2(dec['b2']),
               dec['W1'], r2(dec['bm1']), dec['W2'], r2(dec['bm2']))

    rec_part = _rec(df.reshape(B, N_ROI, (T_LEN // PATCH) * N_EMBD), y_raw,
                    dt['W1'], r2(dt['b1']), dt['W2'], r2(dt['b2']))

    emb_loss = jnp.sum(emb_part) / _f32(ROWS * EMBED_DIM)
    rec_loss = jnp.sum(rec_part) / _f32(B * N_ROI * D_OUT)
    return (emb_loss + rec_loss, ef)
